# baseline (speedup 1.0000x reference)
import functools

import jax
import jax.numpy as jnp
from jax import lax
from jax.experimental import pallas as pl
from jax.experimental.pallas import tpu as pltpu
from jax.experimental.pallas import tpu_sc as plsc

D_MODEL = 128
MAX_LEN = 200
SEQ = 200
BATCH = 4096

_NC = 2
_NS = 16
_NW = _NC * _NS
_K = 16


def _sc_bcast(table_hbm, out_hbm, tab_v, sem):
    wid = lax.axis_index("s") * _NC + lax.axis_index("c")
    per = BATCH // _NW
    base = wid * per
    pltpu.sync_copy(table_hbm, tab_v)

    def chunk(i, carry):
        for j in range(_K):
            pltpu.async_copy(tab_v, out_hbm.at[base + i * _K + j], sem).start()
        for j in range(_K):
            pltpu.make_async_copy(tab_v, out_hbm.at[base + i * _K + j], sem).wait()
        return carry

    lax.fori_loop(0, per // _K, chunk, 0)


def kernel(x, pos_embedding_weight):
    bs, seq = x.shape
    mesh = plsc.VectorSubcoreMesh(core_axis_name="c", subcore_axis_name="s")
    k = functools.partial(
        pl.kernel,
        mesh=mesh,
        out_type=jax.ShapeDtypeStruct((bs, seq, D_MODEL), jnp.float32),
        scratch_types=[
            pltpu.VMEM((seq, D_MODEL), jnp.float32),
            pltpu.SemaphoreType.DMA,
        ],
    )(_sc_bcast)
    return k(pos_embedding_weight[:seq])


# speedup vs baseline: 2.3079x; 2.3079x over previous
import jax
import jax.numpy as jnp
from jax.experimental import pallas as pl

D_MODEL = 128
MAX_LEN = 200
SEQ = 200

_BB = 32


def _bcast_kernel(w_ref, o_ref):
    o_ref[...] = jnp.broadcast_to(w_ref[...][None, :, :], o_ref.shape)


def kernel(x, pos_embedding_weight):
    bs, seq = x.shape
    grid = (bs // _BB,)
    out = pl.pallas_call(
        _bcast_kernel,
        grid=grid,
        in_specs=[pl.BlockSpec((seq, D_MODEL), lambda i: (0, 0))],
        out_specs=pl.BlockSpec((_BB, seq, D_MODEL), lambda i: (i, 0, 0)),
        out_shape=jax.ShapeDtypeStruct((bs, seq, D_MODEL), jnp.float32),
    )(pos_embedding_weight[:seq])
    return out
